# Initial kernel scaffold; baseline (speedup 1.0000x reference)
import jax
import jax.numpy as jnp
from jax.experimental import pallas as pl


def kernel(engagement_id, engagement_features, emb_table, W1, b1, W2, b2, W3, b3):
    raise NotImplementedError("write your pallas kernel here")


# speedup vs baseline: 2.2563x; 2.2563x over previous
import functools

import jax
import jax.numpy as jnp
from jax import lax
from jax.experimental import pallas as pl
from jax.experimental.pallas import tpu as pltpu
from jax.experimental.pallas import tpu_sc as plsc

VOCAB = 1000
EMB = 128
B = 16384
LANES = 16
IDX_CHUNK = 128


def _make_sc_gather():
    info = plsc.get_sparse_core_info()
    nw = info.num_cores * info.num_subcores
    b_per_w = B // nw
    n_chunks = b_per_w // IDX_CHUNK

    mesh = plsc.VectorSubcoreMesh(core_axis_name="c", subcore_axis_name="s")

    @functools.partial(
        pl.kernel,
        mesh=mesh,
        out_type=jax.ShapeDtypeStruct((B, EMB), jnp.float32),
        scratch_types=[
            pltpu.VMEM((b_per_w,), jnp.int32),
            pltpu.VMEM((n_chunks, IDX_CHUNK), jnp.int32),
            pltpu.VMEM((b_per_w, EMB), jnp.float32),
            pltpu.SemaphoreType.DMA,
        ],
    )
    def sc_gather(ids_hbm, table_hbm, out_hbm, ids_v, idx_v, rows_v, sem):
        wid = lax.axis_index("s") * info.num_cores + lax.axis_index("c")
        base = wid * b_per_w
        pltpu.sync_copy(ids_hbm.at[pl.ds(base, b_per_w)], ids_v)
        for j in range(n_chunks):
            for g in range(IDX_CHUNK // LANES):
                raw = ids_v[pl.ds(j * IDX_CHUNK + g * LANES, LANES)]
                h = raw.astype(jnp.uint32) * jnp.uint32(2654435761)
                idx_v[j, pl.ds(g * LANES, LANES)] = (
                    h % jnp.uint32(VOCAB)).astype(jnp.int32)
        copies = [
            pltpu.async_copy(
                table_hbm.at[idx_v.at[j]],
                rows_v.at[pl.ds(j * IDX_CHUNK, IDX_CHUNK)],
                sem,
            )
            for j in range(n_chunks)
        ]
        for c in copies:
            c.wait()
        pltpu.sync_copy(rows_v, out_hbm.at[pl.ds(base, b_per_w)])

    return sc_gather


_sc_gather = _make_sc_gather()

_BLK = 2048


def _tc_body(base_ref, f_ref, w1_ref, b1_ref, w2_ref, b2_ref, w3_ref, b3_ref,
             o_ref):
    h = jnp.maximum(
        jnp.dot(f_ref[...], w1_ref[...], preferred_element_type=jnp.float32)
        + b1_ref[...], 0.0)
    h = jnp.maximum(
        jnp.dot(h, w2_ref[...], preferred_element_type=jnp.float32)
        + b2_ref[...], 0.0)
    feat = jnp.maximum(
        jnp.dot(h, w3_ref[...], preferred_element_type=jnp.float32)
        + b3_ref[...], 0.0)
    c = base_ref[...] + feat
    sq = jnp.sum(c * c, axis=-1, keepdims=True)
    o_ref[...] = c * lax.rsqrt(jnp.maximum(sq, 1e-12))


def _tc_finish(base, feats, W1, b1, W2, b2, W3, b3):
    grid = (B // _BLK,)
    full = lambda shape: pl.BlockSpec(shape, lambda i: (0, 0))
    return pl.pallas_call(
        _tc_body,
        grid=grid,
        in_specs=[
            pl.BlockSpec((_BLK, EMB), lambda i: (i, 0)),
            pl.BlockSpec((_BLK, 10), lambda i: (i, 0)),
            full((10, 64)),
            full((1, 64)),
            full((64, 32)),
            full((1, 32)),
            full((32, EMB)),
            full((1, EMB)),
        ],
        out_specs=pl.BlockSpec((_BLK, EMB), lambda i: (i, 0)),
        out_shape=jax.ShapeDtypeStruct((B, EMB), jnp.float32),
    )(base, feats, W1, b1, W2, b2, W3, b3)


def kernel(engagement_id, engagement_features, emb_table, W1, b1, W2, b2, W3,
           b3):
    base = _sc_gather(engagement_id, emb_table)
    return _tc_finish(base, engagement_features, W1,
                      b1.reshape(1, 64), W2, b2.reshape(1, 32), W3,
                      b3.reshape(1, EMB))


# speedup vs baseline: 2.3442x; 1.0389x over previous
import functools

import jax
import jax.numpy as jnp
from jax import lax
from jax.experimental import pallas as pl
from jax.experimental.pallas import tpu as pltpu
from jax.experimental.pallas import tpu_sc as plsc

VOCAB = 1000
EMB = 128
B = 16384
LANES = 16
IDX_CHUNK = 128


def _make_sc_gather():
    info = plsc.get_sparse_core_info()
    nw = info.num_cores * info.num_subcores
    b_per_w = B // nw
    n_chunks = b_per_w // IDX_CHUNK

    mesh = plsc.VectorSubcoreMesh(core_axis_name="c", subcore_axis_name="s")

    @functools.partial(
        pl.kernel,
        mesh=mesh,
        out_type=jax.ShapeDtypeStruct((B, EMB), jnp.float32),
        scratch_types=[
            pltpu.VMEM((b_per_w,), jnp.int32),
            pltpu.VMEM((n_chunks, IDX_CHUNK), jnp.int32),
            pltpu.VMEM((b_per_w, EMB), jnp.float32),
            pltpu.SemaphoreType.DMA,
            pltpu.SemaphoreType.DMA,
        ],
    )
    def sc_gather(ids_hbm, table_hbm, out_hbm, ids_v, idx_v, rows_v, sem_g,
                  sem_o):
        wid = lax.axis_index("s") * info.num_cores + lax.axis_index("c")
        base = wid * b_per_w
        pltpu.sync_copy(ids_hbm.at[pl.ds(base, b_per_w)], ids_v)
        gathers = []
        for j in range(n_chunks):
            for g in range(IDX_CHUNK // LANES):
                raw = ids_v[pl.ds(j * IDX_CHUNK + g * LANES, LANES)]
                h = raw.astype(jnp.uint32) * jnp.uint32(2654435761)
                idx_v[j, pl.ds(g * LANES, LANES)] = (
                    h % jnp.uint32(VOCAB)).astype(jnp.int32)
            gathers.append(
                pltpu.async_copy(
                    table_hbm.at[idx_v.at[j]],
                    rows_v.at[pl.ds(j * IDX_CHUNK, IDX_CHUNK)],
                    sem_g,
                ))
        outs = []
        for j in range(n_chunks):
            gathers[j].wait()
            outs.append(
                pltpu.async_copy(
                    rows_v.at[pl.ds(j * IDX_CHUNK, IDX_CHUNK)],
                    out_hbm.at[pl.ds(base + j * IDX_CHUNK, IDX_CHUNK)],
                    sem_o,
                ))
        for c in outs:
            c.wait()

    return sc_gather


_sc_gather = _make_sc_gather()

_BLK = 4096


def _tc_body(base_ref, f_ref, w1_ref, b1_ref, w2_ref, b2_ref, w3_ref, b3_ref,
             o_ref):
    h = jnp.maximum(
        jnp.dot(f_ref[...], w1_ref[...], preferred_element_type=jnp.float32)
        + b1_ref[...], 0.0)
    h = jnp.maximum(
        jnp.dot(h, w2_ref[...], preferred_element_type=jnp.float32)
        + b2_ref[...], 0.0)
    feat = jnp.maximum(
        jnp.dot(h, w3_ref[...], preferred_element_type=jnp.float32)
        + b3_ref[...], 0.0)
    c = base_ref[...] + feat
    sq = jnp.sum(c * c, axis=-1, keepdims=True)
    o_ref[...] = c * lax.rsqrt(jnp.maximum(sq, 1e-12))


def _tc_finish(base, feats, W1, b1, W2, b2, W3, b3):
    grid = (B // _BLK,)
    full = lambda shape: pl.BlockSpec(shape, lambda i: (0, 0))
    return pl.pallas_call(
        _tc_body,
        grid=grid,
        in_specs=[
            pl.BlockSpec((_BLK, EMB), lambda i: (i, 0)),
            pl.BlockSpec((_BLK, 10), lambda i: (i, 0)),
            full((10, 64)),
            full((1, 64)),
            full((64, 32)),
            full((1, 32)),
            full((32, EMB)),
            full((1, EMB)),
        ],
        out_specs=pl.BlockSpec((_BLK, EMB), lambda i: (i, 0)),
        out_shape=jax.ShapeDtypeStruct((B, EMB), jnp.float32),
    )(base, feats, W1, b1, W2, b2, W3, b3)


def kernel(engagement_id, engagement_features, emb_table, W1, b1, W2, b2, W3,
           b3):
    base = _sc_gather(engagement_id, emb_table)
    return _tc_finish(base, engagement_features, W1,
                      b1.reshape(1, 64), W2, b2.reshape(1, 32), W3,
                      b3.reshape(1, EMB))
